# Initial kernel scaffold; baseline (speedup 1.0000x reference)
import jax
import jax.numpy as jnp
from jax.experimental import pallas as pl


def kernel(node_features, pos, edge_index, W1, b1, W2, b2, W_tp):
    raise NotImplementedError("write your pallas kernel here")


# speedup vs baseline: 22.4133x; 22.4133x over previous
import functools

import jax
import jax.numpy as jnp
from jax import lax
from jax.experimental import pallas as pl
from jax.experimental.pallas import tpu as pltpu
from jax.experimental.pallas import tpu_sc as plsc

MAX_RADIUS = 3.0
NUM_BASIS = 10
NOUT = 3
NC, NS = 2, 16
NW = NC * NS
LANES = 16

_F32 = jnp.float32
_I32 = jnp.int32


def _edge_vec_call(pos_flat, edge_index, n_nodes, n_edges):
    ch = 2560
    nchunks = n_edges // ch
    nv = ch // LANES

    mesh = plsc.VectorSubcoreMesh(
        core_axis_name="c", subcore_axis_name="s", num_cores=NC, num_subcores=NS
    )

    @functools.partial(
        pl.kernel,
        out_type=tuple(jax.ShapeDtypeStruct((n_edges,), _F32) for _ in range(3)),
        mesh=mesh,
        scratch_types=[
            pltpu.VMEM((3 * n_nodes,), _F32),
            pltpu.VMEM((2, ch), _I32),
            pltpu.VMEM((ch,), _F32),
            pltpu.VMEM((ch,), _F32),
            pltpu.VMEM((ch,), _F32),
        ],
        compiler_params=pltpu.CompilerParams(needs_layout_passes=False),
    )
    def k(pos_hbm, ei_hbm, ox_hbm, oy_hbm, oz_hbm,
          pos_v, ei_v, ex_v, ey_v, ez_v):
        wid = lax.axis_index("s") * NC + lax.axis_index("c")
        pltpu.sync_copy(pos_hbm, pos_v)
        n_mine = (nchunks - wid + NW - 1) // NW

        def chunk_body(i, _):
            cbase = (wid + i * NW) * ch
            pltpu.sync_copy(ei_hbm.at[:, pl.ds(cbase, ch)], ei_v)

            def vec_body(j, _):
                for u in range(5):
                    off = (j * 5 + u) * LANES
                    s3 = ei_v[0, pl.ds(off, LANES)] * 3
                    d3 = ei_v[1, pl.ds(off, LANES)] * 3
                    ex_v[pl.ds(off, LANES)] = (
                        plsc.load_gather(pos_v, [s3]) - plsc.load_gather(pos_v, [d3]))
                    ey_v[pl.ds(off, LANES)] = (
                        plsc.load_gather(pos_v, [s3 + 1]) - plsc.load_gather(pos_v, [d3 + 1]))
                    ez_v[pl.ds(off, LANES)] = (
                        plsc.load_gather(pos_v, [s3 + 2]) - plsc.load_gather(pos_v, [d3 + 2]))
                return 0

            lax.fori_loop(0, nv // 5, vec_body, 0)
            pltpu.sync_copy(ex_v, ox_hbm.at[pl.ds(cbase, ch)])
            pltpu.sync_copy(ey_v, oy_hbm.at[pl.ds(cbase, ch)])
            pltpu.sync_copy(ez_v, oz_hbm.at[pl.ds(cbase, ch)])
            return 0

        lax.fori_loop(0, n_mine, chunk_body, 0)

    return k(pos_flat, edge_index)


def _coeff_body(ex_ref, ey_ref, ez_ref, w1t_ref, b1_ref, w2ct_ref, b2c_ref,
                c0_ref, c1_ref, c2_ref):
    x = ex_ref[...]
    y = ey_ref[...]
    z = ez_ref[...]
    r = jnp.sqrt(x * x + y * y + z * z + 1e-12)

    step = MAX_RADIUS / (NUM_BASIS + 1)
    amp = 1.14136 * (2.718281828459045 ** 2) * (NUM_BASIS ** 0.5)
    rows = []
    for kb in range(NUM_BASIS):
        d = (r - step * (kb + 1)) / step
        inside = jnp.abs(d) < 1.0
        safe = jnp.where(inside, d, 0.0)
        rows.append(jnp.where(inside, amp * jnp.exp(1.0 / (safe * safe - 1.0)), 0.0))
    emb = jnp.stack(rows, axis=0)

    h = lax.dot_general(w1t_ref[...], emb, (((1,), (0,)), ((), ())),
                        precision=lax.Precision.DEFAULT,
                        preferred_element_type=_F32)
    h = h + b1_ref[...][:, None]
    h = h / (1.0 + jnp.exp(-h))
    w27 = lax.dot_general(w2ct_ref[...], h, (((1,), (0,)), ((), ())),
                          precision=lax.Precision.DEFAULT,
                          preferred_element_type=_F32)
    w27 = w27 + b2c_ref[...][:, None]

    inv_r = 1.0 / r
    ux = x * inv_r
    uy = y * inv_r
    uz = z * inv_r
    s3c = 3.0 ** 0.5
    s15 = 15.0 ** 0.5
    s5 = 5.0 ** 0.5
    sh = [jnp.ones_like(ux), s3c * uy, s3c * uz, s3c * ux,
          s15 * ux * uy, s15 * uy * uz, (s5 / 2.0) * (3.0 * uz * uz - 1.0),
          s15 * ux * uz, (s15 / 2.0) * (ux * ux - uy * uy)]

    t = 10.0 * (1.0 - r / MAX_RADIUS)
    cut = jnp.where(t > 0.0, jnp.exp(-1.0 / jnp.where(t > 0.0, t, 1.0)), 0.0)

    for o, out_ref in enumerate((c0_ref, c1_ref, c2_ref)):
        acc = sh[0] * w27[o, :]
        for s in range(1, 9):
            acc = acc + sh[s] * w27[3 * s + o, :]
        out_ref[...] = cut * acc


def _coeff_call(ex, ey, ez, w1t, b1, w2ct, b2c, n_edges):
    tblk = 16384
    grid = ((n_edges + tblk - 1) // tblk,)
    ev_spec = pl.BlockSpec((tblk,), lambda i: (i,))
    full2 = lambda shape: pl.BlockSpec(shape, lambda i: (0,) * len(shape))
    return pl.pallas_call(
        _coeff_body,
        grid=grid,
        in_specs=[ev_spec, ev_spec, ev_spec,
                  full2(w1t.shape), full2(b1.shape),
                  full2(w2ct.shape), full2(b2c.shape)],
        out_specs=[ev_spec, ev_spec, ev_spec],
        out_shape=tuple(jax.ShapeDtypeStruct((n_edges,), _F32) for _ in range(3)),
    )(ex, ey, ez, w1t, b1, w2ct, b2c)


def _scatter_call(xp_flat, edge_index, c0, c1, c2, n_nodes, n_edges):
    ch = 2560
    nchunks = n_edges // ch
    nv = ch // LANES
    acc_len = 3 * n_nodes

    mesh = plsc.VectorSubcoreMesh(
        core_axis_name="c", subcore_axis_name="s", num_cores=NC, num_subcores=NS
    )

    @functools.partial(
        pl.kernel,
        out_type=jax.ShapeDtypeStruct((NW, acc_len), _F32),
        mesh=mesh,
        scratch_types=[
            pltpu.VMEM((acc_len,), _F32),
            pltpu.VMEM((acc_len,), _F32),
            pltpu.VMEM((2, ch), _I32),
            pltpu.VMEM((ch,), _F32),
            pltpu.VMEM((ch,), _F32),
            pltpu.VMEM((ch,), _F32),
        ],
        compiler_params=pltpu.CompilerParams(needs_layout_passes=False),
    )
    def k(xp_hbm, ei_hbm, c0_hbm, c1_hbm, c2_hbm, out_hbm,
          xp_v, acc_v, ei_v, c0_v, c1_v, c2_v):
        wid = lax.axis_index("s") * NC + lax.axis_index("c")
        pltpu.sync_copy(xp_hbm, xp_v)

        zeros16 = jnp.zeros((LANES,), _F32)

        def zero_body(i, _):
            acc_v[pl.ds(i * LANES, LANES)] = zeros16
            return 0

        lax.fori_loop(0, acc_len // LANES, zero_body, 0)

        n_mine = (nchunks - wid + NW - 1) // NW

        def chunk_body(i, _):
            cbase = (wid + i * NW) * ch
            pltpu.sync_copy(ei_hbm.at[:, pl.ds(cbase, ch)], ei_v)
            pltpu.sync_copy(c0_hbm.at[pl.ds(cbase, ch)], c0_v)
            pltpu.sync_copy(c1_hbm.at[pl.ds(cbase, ch)], c1_v)
            pltpu.sync_copy(c2_hbm.at[pl.ds(cbase, ch)], c2_v)

            def vec_body(j, _):
                for u in range(5):
                    off = (j * 5 + u) * LANES
                    s16 = ei_v[0, pl.ds(off, LANES)]
                    d16 = ei_v[1, pl.ds(off, LANES)]
                    v0 = plsc.load_gather(xp_v, [s16]) * c0_v[pl.ds(off, LANES)]
                    plsc.addupdate_scatter(acc_v, [d16], v0)
                    v1 = plsc.load_gather(xp_v, [s16 + n_nodes]) * c1_v[pl.ds(off, LANES)]
                    plsc.addupdate_scatter(acc_v, [d16 + n_nodes], v1)
                    v2 = plsc.load_gather(xp_v, [s16 + 2 * n_nodes]) * c2_v[pl.ds(off, LANES)]
                    plsc.addupdate_scatter(acc_v, [d16 + 2 * n_nodes], v2)
                return 0

            lax.fori_loop(0, nv // 5, vec_body, 0)
            return 0

        lax.fori_loop(0, n_mine, chunk_body, 0)
        pltpu.sync_copy(acc_v, out_hbm.at[wid])

    return k(xp_flat, edge_index, c0, c1, c2)


def _xproj_body(feat_ref, w3t_ref, out_ref):
    out_ref[...] = lax.dot_general(
        w3t_ref[...], feat_ref[...], (((1,), (1,)), ((), ())),
        precision=lax.Precision.DEFAULT, preferred_element_type=_F32)


def _finish_body(part_ref, out_ref):
    psi0 = jnp.sum(part_ref[:, 0, :], axis=0)
    psi1 = jnp.sum(part_ref[:, 1, :], axis=0)
    psi2 = jnp.sum(part_ref[:, 2, :], axis=0)
    out_ref[0, :] = psi2 - psi1
    out_ref[1, :] = psi0 - psi2
    out_ref[2, :] = psi1 - psi0


def kernel(node_features, pos, edge_index, W1, b1, W2, b2, W_tp):
    B, N, F = node_features.shape
    E = edge_index.shape[1]
    out_dim = W_tp.shape[1]

    feat = node_features.reshape(B * N, F)
    pos_flat = pos.reshape(-1)
    ei = edge_index.astype(_I32)

    w3t = W_tp[:, :NOUT].T
    col_idx = jnp.array(
        [s * out_dim + o for s in range(9) for o in range(NOUT)], dtype=_I32)
    w2ct = W2[:, col_idx].T
    b2c = b2[col_idx]
    w1t = W1.T

    xpT = pl.pallas_call(
        _xproj_body,
        out_shape=jax.ShapeDtypeStruct((NOUT, B * N), _F32),
    )(feat, w3t)
    xp_flat = xpT.reshape(-1)

    ex, ey, ez = _edge_vec_call(pos_flat, ei, B * N, E)

    c0, c1, c2 = _coeff_call(ex, ey, ez, w1t, b1, w2ct, b2c, E)

    partials = _scatter_call(xp_flat, ei, c0, c1, c2, B * N, E)

    out3 = pl.pallas_call(
        _finish_body,
        out_shape=jax.ShapeDtypeStruct((3, B * N), _F32),
    )(partials.reshape(NW, 3, B * N))

    return out3.T.reshape(B, N, 3)


# speedup vs baseline: 25.0454x; 1.1174x over previous
import functools

import jax
import jax.numpy as jnp
from jax import lax
from jax.experimental import pallas as pl
from jax.experimental.pallas import tpu as pltpu
from jax.experimental.pallas import tpu_sc as plsc

MAX_RADIUS = 3.0
NUM_BASIS = 10
NOUT = 3
NC, NS = 2, 16
NW = NC * NS
LANES = 16

_F32 = jnp.float32
_I32 = jnp.int32


def _edge_vec_call(pos_flat, edge_index, n_nodes, n_edges, c_lo, c_hi):
    ch = 2560
    nv = ch // LANES

    mesh = plsc.VectorSubcoreMesh(
        core_axis_name="c", subcore_axis_name="s", num_cores=NC, num_subcores=NS
    )

    @functools.partial(
        pl.kernel,
        out_type=tuple(jax.ShapeDtypeStruct((n_edges,), _F32) for _ in range(3)),
        mesh=mesh,
        scratch_types=[
            pltpu.VMEM((3 * n_nodes,), _F32),
            pltpu.VMEM((2, ch), _I32),
            pltpu.VMEM((ch,), _F32),
            pltpu.VMEM((ch,), _F32),
            pltpu.VMEM((ch,), _F32),
        ],
        compiler_params=pltpu.CompilerParams(needs_layout_passes=False),
    )
    def k(pos_hbm, ei_hbm, ox_hbm, oy_hbm, oz_hbm,
          pos_v, ei_v, ex_v, ey_v, ez_v):
        wid = lax.axis_index("s") * NC + lax.axis_index("c")
        pltpu.sync_copy(pos_hbm, pos_v)
        n_mine = (c_hi - c_lo - wid + NW - 1) // NW

        def chunk_body(i, _):
            cbase = (c_lo + wid + i * NW) * ch
            pltpu.sync_copy(ei_hbm.at[:, pl.ds(cbase, ch)], ei_v)

            def vec_body(j, _):
                for u in range(5):
                    off = (j * 5 + u) * LANES
                    s3 = ei_v[0, pl.ds(off, LANES)] * 3
                    d3 = ei_v[1, pl.ds(off, LANES)] * 3
                    ex_v[pl.ds(off, LANES)] = (
                        plsc.load_gather(pos_v, [s3]) - plsc.load_gather(pos_v, [d3]))
                    ey_v[pl.ds(off, LANES)] = (
                        plsc.load_gather(pos_v, [s3 + 1]) - plsc.load_gather(pos_v, [d3 + 1]))
                    ez_v[pl.ds(off, LANES)] = (
                        plsc.load_gather(pos_v, [s3 + 2]) - plsc.load_gather(pos_v, [d3 + 2]))
                return 0

            lax.fori_loop(0, nv // 5, vec_body, 0)
            pltpu.sync_copy(ex_v, ox_hbm.at[pl.ds(cbase, ch)])
            pltpu.sync_copy(ey_v, oy_hbm.at[pl.ds(cbase, ch)])
            pltpu.sync_copy(ez_v, oz_hbm.at[pl.ds(cbase, ch)])
            return 0

        lax.fori_loop(0, n_mine, chunk_body, 0)

    return k(pos_flat, edge_index)


def _coeff_body(ex_ref, ey_ref, ez_ref, w1t_ref, b1_ref, w2ct_ref, b2c_ref,
                c0_ref, c1_ref, c2_ref):
    x = ex_ref[...]
    y = ey_ref[...]
    z = ez_ref[...]
    r = jnp.sqrt(x * x + y * y + z * z + 1e-12)

    step = MAX_RADIUS / (NUM_BASIS + 1)
    amp = 1.14136 * (2.718281828459045 ** 2) * (NUM_BASIS ** 0.5)
    kk = jnp.arange(1, NUM_BASIS + 1, dtype=jnp.int32).astype(_F32)
    centers = (kk * step)[:, None]
    d = (r[None, :] - centers) / step
    inside = jnp.abs(d) < 1.0
    safe = jnp.where(inside, d, 0.0)
    emb = jnp.where(inside, amp * jnp.exp(1.0 / (safe * safe - 1.0)), 0.0)

    h = lax.dot_general(w1t_ref[...], emb, (((1,), (0,)), ((), ())),
                        precision=lax.Precision.DEFAULT,
                        preferred_element_type=_F32)
    h = h + b1_ref[...][:, None]
    h = h / (1.0 + jnp.exp(-h))
    w48 = lax.dot_general(w2ct_ref[...], h, (((1,), (0,)), ((), ())),
                          precision=lax.Precision.DEFAULT,
                          preferred_element_type=_F32)
    w48 = w48 + b2c_ref[...][:, None]

    inv_r = 1.0 / r
    ux = x * inv_r
    uy = y * inv_r
    uz = z * inv_r
    s3c = 3.0 ** 0.5
    s15 = 15.0 ** 0.5
    s5 = 5.0 ** 0.5
    one = jnp.ones_like(ux)
    zero = jnp.zeros_like(ux)
    sh16 = jnp.stack(
        [one, s3c * uy, s3c * uz, s3c * ux,
         s15 * ux * uy, s15 * uy * uz, (s5 / 2.0) * (3.0 * uz * uz - 1.0),
         s15 * ux * uz, (s15 / 2.0) * (ux * ux - uy * uy),
         zero, zero, zero, zero, zero, zero, zero], axis=0)

    t = 10.0 * (1.0 - r / MAX_RADIUS)
    cut = jnp.where(t > 0.0, jnp.exp(-1.0 / jnp.where(t > 0.0, t, 1.0)), 0.0)

    for o, out_ref in enumerate((c0_ref, c1_ref, c2_ref)):
        acc = jnp.sum(sh16 * w48[16 * o:16 * (o + 1), :], axis=0)
        out_ref[...] = cut * acc


def _coeff_call(ex, ey, ez, w1t, b1, w2ct, b2c, n_edges, blk_lo, nblk):
    tblk = 16384
    grid = (nblk,)
    ev_spec = pl.BlockSpec((tblk,), lambda i: (i + blk_lo,))
    full2 = lambda shape: pl.BlockSpec(shape, lambda i: (0,) * len(shape))
    return pl.pallas_call(
        _coeff_body,
        grid=grid,
        in_specs=[ev_spec, ev_spec, ev_spec,
                  full2(w1t.shape), full2(b1.shape),
                  full2(w2ct.shape), full2(b2c.shape)],
        out_specs=[ev_spec, ev_spec, ev_spec],
        out_shape=tuple(jax.ShapeDtypeStruct((n_edges,), _F32) for _ in range(3)),
    )(ex, ey, ez, w1t, b1, w2ct, b2c)


def _scatter_call(xp_flat, edge_index, c0, c1, c2, n_nodes, n_edges, c_lo, c_hi):
    ch = 2560
    nv = ch // LANES
    acc_len = 3 * n_nodes

    mesh = plsc.VectorSubcoreMesh(
        core_axis_name="c", subcore_axis_name="s", num_cores=NC, num_subcores=NS
    )

    @functools.partial(
        pl.kernel,
        out_type=jax.ShapeDtypeStruct((NW, acc_len), _F32),
        mesh=mesh,
        scratch_types=[
            pltpu.VMEM((acc_len,), _F32),
            pltpu.VMEM((acc_len,), _F32),
            pltpu.VMEM((2, ch), _I32),
            pltpu.VMEM((ch,), _F32),
            pltpu.VMEM((ch,), _F32),
            pltpu.VMEM((ch,), _F32),
        ],
        compiler_params=pltpu.CompilerParams(needs_layout_passes=False),
    )
    def k(xp_hbm, ei_hbm, c0_hbm, c1_hbm, c2_hbm, out_hbm,
          xp_v, acc_v, ei_v, c0_v, c1_v, c2_v):
        wid = lax.axis_index("s") * NC + lax.axis_index("c")
        pltpu.sync_copy(xp_hbm, xp_v)

        zeros16 = jnp.zeros((LANES,), _F32)

        def zero_body(i, _):
            acc_v[pl.ds(i * LANES, LANES)] = zeros16
            return 0

        lax.fori_loop(0, acc_len // LANES, zero_body, 0)

        n_mine = (c_hi - c_lo - wid + NW - 1) // NW

        def chunk_body(i, _):
            cbase = (c_lo + wid + i * NW) * ch
            pltpu.sync_copy(ei_hbm.at[:, pl.ds(cbase, ch)], ei_v)
            pltpu.sync_copy(c0_hbm.at[pl.ds(cbase, ch)], c0_v)
            pltpu.sync_copy(c1_hbm.at[pl.ds(cbase, ch)], c1_v)
            pltpu.sync_copy(c2_hbm.at[pl.ds(cbase, ch)], c2_v)

            def vec_body(j, _):
                for u in range(5):
                    off = (j * 5 + u) * LANES
                    s16 = ei_v[0, pl.ds(off, LANES)]
                    d16 = ei_v[1, pl.ds(off, LANES)]
                    v0 = plsc.load_gather(xp_v, [s16]) * c0_v[pl.ds(off, LANES)]
                    plsc.addupdate_scatter(acc_v, [d16], v0)
                    v1 = plsc.load_gather(xp_v, [s16 + n_nodes]) * c1_v[pl.ds(off, LANES)]
                    plsc.addupdate_scatter(acc_v, [d16 + n_nodes], v1)
                    v2 = plsc.load_gather(xp_v, [s16 + 2 * n_nodes]) * c2_v[pl.ds(off, LANES)]
                    plsc.addupdate_scatter(acc_v, [d16 + 2 * n_nodes], v2)
                return 0

            lax.fori_loop(0, nv // 5, vec_body, 0)
            return 0

        lax.fori_loop(0, n_mine, chunk_body, 0)
        pltpu.sync_copy(acc_v, out_hbm.at[wid])

    return k(xp_flat, edge_index, c0, c1, c2)


def _xproj_body(feat_ref, w3t_ref, out_ref):
    out_ref[...] = lax.dot_general(
        w3t_ref[...], feat_ref[...], (((1,), (1,)), ((), ())),
        precision=lax.Precision.DEFAULT, preferred_element_type=_F32)


def _finish_body(pa_ref, pb_ref, out_ref):
    psi0 = jnp.sum(pa_ref[:, 0, :], axis=0) + jnp.sum(pb_ref[:, 0, :], axis=0)
    psi1 = jnp.sum(pa_ref[:, 1, :], axis=0) + jnp.sum(pb_ref[:, 1, :], axis=0)
    psi2 = jnp.sum(pa_ref[:, 2, :], axis=0) + jnp.sum(pb_ref[:, 2, :], axis=0)
    out_ref[0, :] = psi2 - psi1
    out_ref[1, :] = psi0 - psi2
    out_ref[2, :] = psi1 - psi0


def kernel(node_features, pos, edge_index, W1, b1, W2, b2, W_tp):
    B, N, F = node_features.shape
    E = edge_index.shape[1]
    out_dim = W_tp.shape[1]

    feat = node_features.reshape(B * N, F)
    pos_flat = pos.reshape(-1)
    ei = edge_index.astype(_I32)

    w3t = W_tp[:, :NOUT].T
    col_idx = jnp.array(
        [s * out_dim + o for o in range(NOUT) for s in range(9)], dtype=_I32)
    w2ct27 = W2[:, col_idx].T
    b2c27 = b2[col_idx]
    rows48 = jnp.array(
        [o * 16 + s for o in range(NOUT) for s in range(9)], dtype=_I32)
    w2ct = jnp.zeros((48, W2.shape[0]), _F32).at[rows48].set(w2ct27)
    b2c = jnp.zeros((48,), _F32).at[rows48].set(b2c27)
    w1t = W1.T

    xpT = pl.pallas_call(
        _xproj_body,
        out_shape=jax.ShapeDtypeStruct((NOUT, B * N), _F32),
    )(feat, w3t)
    xp_flat = xpT.reshape(-1)

    ch = 2560
    nchunks = E // ch
    split = 64
    nblk_a = (split * ch) // 16384
    nblk_b = (E - split * ch + 16383) // 16384

    exa, eya, eza = _edge_vec_call(pos_flat, ei, B * N, E, 0, split)
    ca0, ca1, ca2 = _coeff_call(exa, eya, eza, w1t, b1, w2ct, b2c, E, 0, nblk_a)
    exb, eyb, ezb = _edge_vec_call(pos_flat, ei, B * N, E, split, nchunks)
    cb0, cb1, cb2 = _coeff_call(exb, eyb, ezb, w1t, b1, w2ct, b2c, E, nblk_a, nblk_b)
    pa = _scatter_call(xp_flat, ei, ca0, ca1, ca2, B * N, E, 0, split)
    pb = _scatter_call(xp_flat, ei, cb0, cb1, cb2, B * N, E, split, nchunks)

    out3 = pl.pallas_call(
        _finish_body,
        out_shape=jax.ShapeDtypeStruct((3, B * N), _F32),
    )(pa.reshape(NW, 3, B * N), pb.reshape(NW, 3, B * N))

    return out3.T.reshape(B, N, 3)
